# baseline (speedup 1.0000x reference)
import functools

import jax
import jax.numpy as jnp
from jax import lax
from jax.experimental import pallas as pl
from jax.experimental.pallas import tpu as pltpu
from jax.experimental.pallas import tpu_sc as plsc

_NC = 2
_NS = 16


def _unfold(x):
    B, C, H, W = x.shape
    xp = jnp.pad(x, ((0, 0), (0, 0), (1, 1), (1, 1)))
    patches = [xp[:, :, ki:ki + H, kj:kj + W] for ki in range(3) for kj in range(3)]
    p = jnp.stack(patches, axis=2)
    return p.reshape(B, C * 9, H * W)


def _topk_body(xbT_ref, idx_ref, *, TILE, HW, C):
    t = pl.program_id(1)

    xbT = xbT_ref[0]
    rows = xbT_ref[0, pl.ds(t * TILE, TILE), :]
    Rt = jax.lax.dot_general(xbT, rows.T, (((1,), (0,)), ((), ())),
                             preferred_element_type=jnp.float32)

    coln = jax.lax.broadcasted_iota(jnp.int32, (HW, TILE), 0)
    idxs = []
    v = Rt
    for j in range(3):
        m = jnp.max(v, axis=0, keepdims=True)
        idx = jnp.min(jnp.where(v >= m, coln, HW), axis=0, keepdims=True)
        idxs.append(idx)
        if j < 2:
            v = jnp.where(coln == idx, -jnp.inf, v)
    idx_ref[0, 0] = jnp.concatenate(idxs, axis=0)


def _attn_body(p_ref, q_ref, out_ref, *, C):
    q = q_ref[0]
    scale = 1.0 / (C ** 0.5)
    ps = []
    sc_list = []
    for j in range(3):
        for a in range(9):
            pj = p_ref[0, :, j, a * C:(a + 1) * C].astype(jnp.float32)
            ps.append(pj)
            sc_list.append(jnp.sum(pj * q, axis=1, keepdims=True) * scale)
    S = jnp.concatenate(sc_list, axis=1)
    m = jnp.max(S, axis=1, keepdims=True)
    e = jnp.exp(S - m)
    w = e / jnp.sum(e, axis=1, keepdims=True)
    acc = jnp.zeros(q.shape, jnp.float32)
    for k in range(27):
        acc = acc + w[:, k:k + 1] * ps[k]
    out_ref[0] = acc


def _sc_gather(table, gidx):
    NR, WPR = table.shape
    NG = gidx.shape[0]
    NW = _NC * _NS
    WROWS = NG // NW
    G = 72
    NCH = WROWS // G
    mesh = plsc.VectorSubcoreMesh(core_axis_name="c", subcore_axis_name="s")

    @functools.partial(
        pl.kernel, mesh=mesh,
        out_type=jax.ShapeDtypeStruct((NG, WPR), jnp.int32),
        scratch_types=[
            pltpu.VMEM((WROWS,), jnp.int32),
            pltpu.VMEM((G, WPR), jnp.int32),
            pltpu.SemaphoreType.DMA,
        ],
    )
    def k(table_hbm, idx_hbm, out_hbm, idx_v, rows_v, sem):
        wid = lax.axis_index("s") * _NC + lax.axis_index("c")
        base = wid * WROWS
        pltpu.sync_copy(idx_hbm.at[pl.ds(base, WROWS)], idx_v)
        for c in range(NCH):
            pltpu.async_copy(
                table_hbm.at[idx_v.at[pl.ds(c * G, G)]], rows_v, sem).wait()
            pltpu.sync_copy(rows_v, out_hbm.at[pl.ds(base + c * G, G)])

    return k(table, gidx)


@jax.jit
def kernel(x):
    B, C, H, W = x.shape
    HW = H * W
    F = 9 * C
    xu = _unfold(x)
    norm = jnp.sqrt(jnp.sum(xu * xu, axis=2, keepdims=True))
    xun = xu / jnp.maximum(norm, 1e-12)
    xb = xun.astype(jnp.bfloat16)
    xbT = xb.transpose(0, 2, 1)
    q = x.reshape(B, C, HW).transpose(0, 2, 1)

    TILE = 256 if HW % 256 == 0 else HW
    NT = HW // TILE
    grid = (B, NT)

    idx = pl.pallas_call(
        functools.partial(_topk_body, TILE=TILE, HW=HW, C=C),
        grid=grid,
        in_specs=[pl.BlockSpec((1, HW, F), lambda b, t: (b, 0, 0))],
        out_specs=pl.BlockSpec((1, 1, 3, TILE), lambda b, t: (b, t, 0, 0)),
        out_shape=jax.ShapeDtypeStruct((B, NT, 3, TILE), jnp.int32),
        compiler_params=pltpu.CompilerParams(
            dimension_semantics=("parallel", "arbitrary"),
        ),
    )(xbT)

    gidx = idx.transpose(0, 1, 3, 2).reshape(B, HW, 3)
    gidx = gidx + (jnp.arange(B, dtype=jnp.int32) * HW)[:, None, None]
    gidx = gidx.reshape(B * HW * 3)

    WPR = F // 2
    WPAD = (-WPR) % 128
    table = jax.lax.bitcast_convert_type(
        xbT.reshape(B * HW, WPR, 2), jnp.int32)
    table = jnp.pad(table, ((0, 0), (0, WPAD)))
    rows = _sc_gather(table, gidx)

    patches = jax.lax.bitcast_convert_type(rows, jnp.bfloat16)
    patches = patches.reshape(B, HW, 3, 2 * (WPR + WPAD))

    out = pl.pallas_call(
        functools.partial(_attn_body, C=C),
        grid=grid,
        in_specs=[
            pl.BlockSpec((1, TILE, 3, patches.shape[-1]),
                         lambda b, t: (b, t, 0, 0)),
            pl.BlockSpec((1, TILE, C), lambda b, t: (b, t, 0)),
        ],
        out_specs=pl.BlockSpec((1, TILE, C), lambda b, t: (b, t, 0)),
        out_shape=jax.ShapeDtypeStruct((B, HW, C), jnp.float32),
        compiler_params=pltpu.CompilerParams(
            dimension_semantics=("parallel", "arbitrary"),
        ),
    )(patches, q)
    return out


# speedup vs baseline: 13.8318x; 13.8318x over previous
import functools

import jax
import jax.numpy as jnp
from jax import lax
from jax.experimental import pallas as pl
from jax.experimental.pallas import tpu as pltpu
from jax.experimental.pallas import tpu_sc as plsc

_NC = 2
_NS = 16


def _unfold(x):
    B, C, H, W = x.shape
    xp = jnp.pad(x, ((0, 0), (0, 0), (1, 1), (1, 1)))
    patches = [xp[:, :, ki:ki + H, kj:kj + W] for ki in range(3) for kj in range(3)]
    p = jnp.stack(patches, axis=2)
    return p.reshape(B, C * 9, H * W)


def _topk_body(xunT_ref, idx_ref, *, TILE, HW, C):
    t = pl.program_id(1)

    xbT = xunT_ref[0].astype(jnp.bfloat16)
    rows = xunT_ref[0, pl.ds(t * TILE, TILE), :].astype(jnp.bfloat16)
    Rt = jax.lax.dot_general(xbT, rows.T, (((1,), (0,)), ((), ())),
                             preferred_element_type=jnp.float32)

    coln = jax.lax.broadcasted_iota(jnp.int32, (HW, TILE), 0)
    idxs = []
    v = Rt
    for j in range(3):
        m = jnp.max(v, axis=0, keepdims=True)
        idx = jnp.min(jnp.where(v >= m, coln, HW), axis=0, keepdims=True)
        idxs.append(idx)
        if j < 2:
            v = jnp.where(coln == idx, -jnp.inf, v)
    idx_ref[0, 0] = jnp.concatenate(idxs, axis=0)


def _attn_body(p_ref, q_ref, out_ref, *, C):
    q = q_ref[0]
    scale = 1.0 / (C ** 0.5)
    ps = []
    sc_list = []
    for j in range(3):
        for a in range(9):
            pj = p_ref[0, :, j, a * C:(a + 1) * C]
            ps.append(pj)
            sc_list.append(jnp.sum(pj * q, axis=1, keepdims=True) * scale)
    S = jnp.concatenate(sc_list, axis=1)
    m = jnp.max(S, axis=1, keepdims=True)
    e = jnp.exp(S - m)
    w = e / jnp.sum(e, axis=1, keepdims=True)
    acc = jnp.zeros(q.shape, jnp.float32)
    for k in range(27):
        acc = acc + w[:, k:k + 1] * ps[k]
    out_ref[0] = acc


def _sc_gather(table, gidx):
    NR, FP = table.shape
    NG = gidx.shape[0]
    NW = _NC * _NS
    WROWS = NG // NW
    G = 48
    NCH = WROWS // G
    mesh = plsc.VectorSubcoreMesh(core_axis_name="c", subcore_axis_name="s")

    @functools.partial(
        pl.kernel, mesh=mesh,
        out_type=jax.ShapeDtypeStruct((NG, FP), jnp.float32),
        scratch_types=[
            pltpu.VMEM((WROWS,), jnp.int32),
            pltpu.VMEM((G, FP), jnp.float32),
            pltpu.SemaphoreType.DMA,
        ],
    )
    def k(table_hbm, idx_hbm, out_hbm, idx_v, rows_v, sem):
        wid = lax.axis_index("s") * _NC + lax.axis_index("c")
        base = wid * WROWS
        pltpu.sync_copy(idx_hbm.at[pl.ds(base, WROWS)], idx_v)
        for c in range(NCH):
            pltpu.async_copy(
                table_hbm.at[idx_v.at[pl.ds(c * G, G)]], rows_v, sem).wait()
            pltpu.sync_copy(rows_v, out_hbm.at[pl.ds(base + c * G, G)])

    return k(table, gidx)


@jax.jit
def kernel(x):
    B, C, H, W = x.shape
    HW = H * W
    F = 9 * C
    xu = _unfold(x)
    norm = jnp.sqrt(jnp.sum(xu * xu, axis=2, keepdims=True))
    xun = xu / jnp.maximum(norm, 1e-12)
    xunT = xun.transpose(0, 2, 1)
    q = x.reshape(B, C, HW).transpose(0, 2, 1)

    TILE = 256 if HW % 256 == 0 else HW
    NT = HW // TILE
    grid = (B, NT)

    idx = pl.pallas_call(
        functools.partial(_topk_body, TILE=TILE, HW=HW, C=C),
        grid=grid,
        in_specs=[pl.BlockSpec((1, HW, F), lambda b, t: (b, 0, 0))],
        out_specs=pl.BlockSpec((1, 1, 3, TILE), lambda b, t: (b, t, 0, 0)),
        out_shape=jax.ShapeDtypeStruct((B, NT, 3, TILE), jnp.int32),
        compiler_params=pltpu.CompilerParams(
            dimension_semantics=("parallel", "arbitrary"),
        ),
    )(xunT)

    gidx = idx.transpose(0, 1, 3, 2).reshape(B, HW, 3)
    gidx = gidx + (jnp.arange(B, dtype=jnp.int32) * HW)[:, None, None]
    gidx = gidx.reshape(B * HW * 3)

    FPAD = (-F) % 128
    table = jnp.pad(xunT.reshape(B * HW, F), ((0, 0), (0, FPAD)))
    rows = _sc_gather(table, gidx)
    patches = rows.reshape(B, HW, 3, F + FPAD)

    out = pl.pallas_call(
        functools.partial(_attn_body, C=C),
        grid=grid,
        in_specs=[
            pl.BlockSpec((1, TILE, 3, patches.shape[-1]),
                         lambda b, t: (b, t, 0, 0)),
            pl.BlockSpec((1, TILE, C), lambda b, t: (b, t, 0)),
        ],
        out_specs=pl.BlockSpec((1, TILE, C), lambda b, t: (b, t, 0)),
        out_shape=jax.ShapeDtypeStruct((B, HW, C), jnp.float32),
        compiler_params=pltpu.CompilerParams(
            dimension_semantics=("parallel", "arbitrary"),
        ),
    )(patches, q)
    return out
